# baseline (speedup 1.0000x reference)
import jax
import jax.numpy as jnp
from jax.experimental import pallas as pl
from jax.experimental.pallas import tpu as pltpu

_EPS = 1.0


def _fused_body(z_ref, mu_ref, mass_ref, w_ref, b_ref,
                fine_ref, coarse_ref, comb_ref):
    z = z_ref[...]
    mu = mu_ref[...]
    mass = mass_ref[...]
    W = w_ref[...]
    b = b_ref[...]

    z2 = jnp.sum(z * z, axis=1, keepdims=True)
    mu2 = jnp.sum(mu * mu, axis=1)[None, :]
    zmu = jax.lax.dot_general(z, mu, (((1,), (1,)), ((), ())),
                              preferred_element_type=jnp.float32)
    d2 = z2 + mu2 - 2.0 * zmu
    w = mass / (d2 + _EPS)
    wmu = jax.lax.dot_general(w, mu, (((1,), (0,)), ((), ())),
                              preferred_element_type=jnp.float32)
    fine = z + wmu - z * jnp.sum(w, axis=1, keepdims=True)

    coarse_raw = jax.lax.dot_general(fine, W, (((1,), (0,)), ((), ())),
                                     preferred_element_type=jnp.float32) + b
    coarse = coarse_raw / (
        jnp.sqrt(jnp.sum(coarse_raw * coarse_raw, axis=1, keepdims=True)) + 1e-8)
    fine_n = fine / (
        jnp.sqrt(jnp.sum(fine * fine, axis=1, keepdims=True)) + 1e-8)

    fine_ref[...] = fine
    coarse_ref[...] = coarse
    comb_ref[...] = jnp.concatenate([fine_n, coarse], axis=1)


@jax.jit
def kernel(z, mu_fine, mass_fine, W_coarse, b_coarse):
    n, d = z.shape
    k = mu_fine.shape[0]
    dc = W_coarse.shape[1]

    block = 4000
    while n % block:
        block //= 2
    grid = (n // block,)

    fine, coarse, combined = pl.pallas_call(
        _fused_body,
        grid=grid,
        in_specs=[
            pl.BlockSpec((block, d), lambda i: (i, 0)),
            pl.BlockSpec((k, d), lambda i: (0, 0)),
            pl.BlockSpec((1, k), lambda i: (0, 0)),
            pl.BlockSpec((d, dc), lambda i: (0, 0)),
            pl.BlockSpec((1, dc), lambda i: (0, 0)),
        ],
        out_specs=[
            pl.BlockSpec((block, d), lambda i: (i, 0)),
            pl.BlockSpec((block, dc), lambda i: (i, 0)),
            pl.BlockSpec((block, d + dc), lambda i: (i, 0)),
        ],
        out_shape=[
            jax.ShapeDtypeStruct((n, d), jnp.float32),
            jax.ShapeDtypeStruct((n, dc), jnp.float32),
            jax.ShapeDtypeStruct((n, d + dc), jnp.float32),
        ],
        compiler_params=pltpu.CompilerParams(
            dimension_semantics=("arbitrary",)),
    )(z, mu_fine, mass_fine.reshape(1, k), W_coarse, b_coarse.reshape(1, dc))
    return fine, coarse, combined


# speedup vs baseline: 1.0471x; 1.0471x over previous
import functools

import jax
import jax.numpy as jnp
from jax.experimental import pallas as pl
from jax.experimental.pallas import tpu as pltpu

_EPS = 1.0


def _dot(a, b):
    return jax.lax.dot_general(a, b, (((1,), (0,)), ((), ())),
                               preferred_element_type=jnp.float32)


def _fused_body(z_ref, m2mu_ref, ones_k_ref, d2c_ref, mass_ref, m2_ref,
                w_ref, b_ref, sel_ref, bc_ref,
                fine_ref, coarse_ref, comb_ref):
    z = z_ref[...]

    zz = z * z
    d2e = _dot(z, m2mu_ref[...]) + _dot(zz, ones_k_ref[...]) + d2c_ref[...]
    w = mass_ref[...] / d2e

    r2 = _dot(w, m2_ref[...])
    d = z.shape[1]
    fine = z * (1.0 - r2[:, d:]) + r2[:, :d]

    cr = _dot(fine, w_ref[...]) + b_ref[...]

    ss = _dot(fine * fine, sel_ref[...][:d]) + _dot(cr * cr, sel_ref[...][d:])
    inv = jax.lax.rsqrt(ss + 1e-30)
    binv = _dot(inv, bc_ref[...])

    comb = jnp.concatenate([fine, cr], axis=1) * binv
    fine_ref[...] = fine
    coarse_ref[...] = comb[:, d:]
    comb_ref[...] = comb


@functools.partial(jax.jit, static_argnames=("block",))
def _run(z, mu_fine, mass_fine, W_coarse, b_coarse, block=4000):
    n, d = z.shape
    k = mu_fine.shape[0]
    dc = W_coarse.shape[1]
    grid = (n // block,)

    mu = mu_fine
    m2mu = (-2.0 * mu).T
    ones_k = jnp.ones((d, k), jnp.float32)
    d2c = (jnp.sum(mu * mu, axis=1) + _EPS)[None, :]
    mass = mass_fine.reshape(1, k)
    m2 = jnp.concatenate([mu, jnp.ones((k, d), jnp.float32)], axis=1)
    sel = jnp.concatenate([
        jnp.tile(jnp.array([[1.0, 0.0]], jnp.float32), (d, 1)),
        jnp.tile(jnp.array([[0.0, 1.0]], jnp.float32), (dc, 1)),
    ], axis=0)
    bc = jnp.concatenate([
        jnp.concatenate([jnp.ones((1, d), jnp.float32),
                         jnp.zeros((1, dc), jnp.float32)], axis=1),
        jnp.concatenate([jnp.zeros((1, d), jnp.float32),
                         jnp.ones((1, dc), jnp.float32)], axis=1),
    ], axis=0)

    full = lambda shape: pl.BlockSpec(shape, lambda i: (0, 0))
    fine, coarse, combined = pl.pallas_call(
        _fused_body,
        grid=grid,
        in_specs=[
            pl.BlockSpec((block, d), lambda i: (i, 0)),
            full((d, k)), full((d, k)), full((1, k)), full((1, k)),
            full((k, 2 * d)), full((d, dc)), full((1, dc)),
            full((d + dc, 2)), full((2, d + dc)),
        ],
        out_specs=[
            pl.BlockSpec((block, d), lambda i: (i, 0)),
            pl.BlockSpec((block, dc), lambda i: (i, 0)),
            pl.BlockSpec((block, d + dc), lambda i: (i, 0)),
        ],
        out_shape=[
            jax.ShapeDtypeStruct((n, d), jnp.float32),
            jax.ShapeDtypeStruct((n, dc), jnp.float32),
            jax.ShapeDtypeStruct((n, d + dc), jnp.float32),
        ],
        compiler_params=pltpu.CompilerParams(
            dimension_semantics=("arbitrary",)),
    )(z, m2mu, ones_k, d2c, mass, m2, W_coarse, b_coarse.reshape(1, dc),
      sel, bc)
    return fine, coarse, combined


def kernel(z, mu_fine, mass_fine, W_coarse, b_coarse):
    return _run(z, mu_fine, mass_fine, W_coarse, b_coarse, block=4000)


# speedup vs baseline: 5.3123x; 5.0732x over previous
import functools

import jax
import jax.numpy as jnp
from jax.experimental import pallas as pl
from jax.experimental.pallas import tpu as pltpu

_EPS = 1.0


def _fused_body(z_ref, nmu_ref, onesk_ref, d2c_ref, mass_ref, m2_ref,
                wt_ref, b_ref, sel_ref, bc_ref,
                fine_ref, coarse_ref, comb_ref):
    zt = z_ref[...]
    d = zt.shape[0]

    d2e = (jnp.dot(nmu_ref[...], zt, preferred_element_type=jnp.float32)
           + jnp.dot(onesk_ref[...], zt * zt,
                     preferred_element_type=jnp.float32)
           + d2c_ref[...])
    w = mass_ref[...] / d2e

    r2 = jnp.dot(m2_ref[...], w, preferred_element_type=jnp.float32)
    fine = zt * (1.0 - r2[d:]) + r2[:d]

    cr = (jnp.dot(wt_ref[...], fine, preferred_element_type=jnp.float32)
          + b_ref[...])

    u = jnp.concatenate([fine * fine, cr * cr], axis=0)
    ss = jnp.dot(sel_ref[...], u, preferred_element_type=jnp.float32)
    inv = jax.lax.rsqrt(ss + 1e-30)
    binv = jnp.dot(bc_ref[...], inv, preferred_element_type=jnp.float32)

    comb = jnp.concatenate([fine, cr], axis=0) * binv
    fine_ref[...] = fine
    coarse_ref[...] = comb[d:]
    comb_ref[...] = comb


@jax.jit
def _run(z, mu_fine, mass_fine, W_coarse, b_coarse):
    n, d = z.shape
    k = mu_fine.shape[0]
    dc = W_coarse.shape[1]
    block = 4096
    grid = (pl.cdiv(n, block),)

    mu = mu_fine
    nmu = -2.0 * mu
    onesk = jnp.ones((k, d), jnp.float32)
    d2c = (jnp.sum(mu * mu, axis=1) + _EPS)[:, None]
    mass = mass_fine[:, None]
    m2 = jnp.concatenate([mu.T, jnp.ones((d, k), jnp.float32)], axis=0)
    sel = jnp.concatenate([
        jnp.concatenate([jnp.ones((1, d), jnp.float32),
                         jnp.zeros((1, dc), jnp.float32)], axis=1),
        jnp.concatenate([jnp.zeros((1, d), jnp.float32),
                         jnp.ones((1, dc), jnp.float32)], axis=1),
    ], axis=0)
    bc = jnp.concatenate([
        jnp.tile(jnp.array([[1.0, 0.0]], jnp.float32), (d, 1)),
        jnp.tile(jnp.array([[0.0, 1.0]], jnp.float32), (dc, 1)),
    ], axis=0)

    full = lambda shape: pl.BlockSpec(shape, lambda i: (0, 0))
    finet, coarset, combt = pl.pallas_call(
        _fused_body,
        grid=grid,
        in_specs=[
            pl.BlockSpec((d, block), lambda i: (0, i)),
            full((k, d)), full((k, d)), full((k, 1)), full((k, 1)),
            full((2 * d, k)), full((dc, d)), full((dc, 1)),
            full((2, d + dc)), full((d + dc, 2)),
        ],
        out_specs=[
            pl.BlockSpec((d, block), lambda i: (0, i)),
            pl.BlockSpec((dc, block), lambda i: (0, i)),
            pl.BlockSpec((d + dc, block), lambda i: (0, i)),
        ],
        out_shape=[
            jax.ShapeDtypeStruct((d, n), jnp.float32),
            jax.ShapeDtypeStruct((dc, n), jnp.float32),
            jax.ShapeDtypeStruct((d + dc, n), jnp.float32),
        ],
        compiler_params=pltpu.CompilerParams(
            dimension_semantics=("arbitrary",)),
    )(z.T, nmu, onesk, d2c, mass, m2, W_coarse.T, b_coarse[:, None],
      sel, bc)
    return finet.T, coarset.T, combt.T


def kernel(z, mu_fine, mass_fine, W_coarse, b_coarse):
    return _run(z, mu_fine, mass_fine, W_coarse, b_coarse)


# speedup vs baseline: 6.8811x; 1.2953x over previous
import functools

import jax
import jax.numpy as jnp
from jax.experimental import pallas as pl
from jax.experimental.pallas import tpu as pltpu

_EPS = 1.0


def _fused_body(z_ref, nmu_ref, onesk_ref, d2c_ref, mass_ref, m2_ref,
                wt_ref, b_ref, sel_ref, bc_ref,
                fine_ref, coarse_ref, comb_ref):
    zt = z_ref[...]
    d = zt.shape[0]

    d2e = (jnp.dot(nmu_ref[...], zt, preferred_element_type=jnp.float32)
           + jnp.dot(onesk_ref[...], zt * zt,
                     preferred_element_type=jnp.float32)
           + d2c_ref[...])
    w = mass_ref[...] / d2e

    r2 = jnp.dot(m2_ref[...], w, preferred_element_type=jnp.float32)
    fine = zt * (1.0 - r2[d:]) + r2[:d]

    cr = (jnp.dot(wt_ref[...], fine, preferred_element_type=jnp.float32)
          + b_ref[...])

    u = jnp.concatenate([fine * fine, cr * cr], axis=0)
    ss = jnp.dot(sel_ref[...], u, preferred_element_type=jnp.float32)
    inv = jax.lax.rsqrt(ss + 1e-30)
    binv = jnp.dot(bc_ref[...], inv, preferred_element_type=jnp.float32)

    comb = jnp.concatenate([fine, cr], axis=0) * binv
    fine_ref[...] = fine
    coarse_ref[...] = comb[d:]
    comb_ref[...] = comb


@jax.jit
def _run(z, mu_fine, mass_fine, W_coarse, b_coarse):
    n, d = z.shape
    k = mu_fine.shape[0]
    dc = W_coarse.shape[1]
    block = 8192
    grid = (pl.cdiv(n, block),)

    mu = mu_fine
    nmu = -2.0 * mu
    onesk = jnp.ones((k, d), jnp.float32)
    d2c = (jnp.sum(mu * mu, axis=1) + _EPS)[:, None]
    mass = mass_fine[:, None]
    m2 = jnp.concatenate([mu.T, jnp.ones((d, k), jnp.float32)], axis=0)
    sel = jnp.concatenate([
        jnp.concatenate([jnp.ones((1, d), jnp.float32),
                         jnp.zeros((1, dc), jnp.float32)], axis=1),
        jnp.concatenate([jnp.zeros((1, d), jnp.float32),
                         jnp.ones((1, dc), jnp.float32)], axis=1),
    ], axis=0)
    bc = jnp.concatenate([
        jnp.tile(jnp.array([[1.0, 0.0]], jnp.float32), (d, 1)),
        jnp.tile(jnp.array([[0.0, 1.0]], jnp.float32), (dc, 1)),
    ], axis=0)

    full = lambda shape: pl.BlockSpec(shape, lambda i: (0, 0))
    finet, coarset, combt = pl.pallas_call(
        _fused_body,
        grid=grid,
        in_specs=[
            pl.BlockSpec((d, block), lambda i: (0, i)),
            full((k, d)), full((k, d)), full((k, 1)), full((k, 1)),
            full((2 * d, k)), full((dc, d)), full((dc, 1)),
            full((2, d + dc)), full((d + dc, 2)),
        ],
        out_specs=[
            pl.BlockSpec((d, block), lambda i: (0, i)),
            pl.BlockSpec((dc, block), lambda i: (0, i)),
            pl.BlockSpec((d + dc, block), lambda i: (0, i)),
        ],
        out_shape=[
            jax.ShapeDtypeStruct((d, n), jnp.float32),
            jax.ShapeDtypeStruct((dc, n), jnp.float32),
            jax.ShapeDtypeStruct((d + dc, n), jnp.float32),
        ],
        compiler_params=pltpu.CompilerParams(
            dimension_semantics=("parallel",)),
    )(z.T, nmu, onesk, d2c, mass, m2, W_coarse.T, b_coarse[:, None],
      sel, bc)
    return finet.T, coarset.T, combt.T


def kernel(z, mu_fine, mass_fine, W_coarse, b_coarse):
    return _run(z, mu_fine, mass_fine, W_coarse, b_coarse)


# speedup vs baseline: 7.6486x; 1.1115x over previous
import functools

import jax
import jax.numpy as jnp
from jax.experimental import pallas as pl
from jax.experimental.pallas import tpu as pltpu

_EPS = 1.0


def _fused_body(z_ref, nmu_ref, onesk_ref, d2c_ref, mass_ref, m2_ref,
                wt_ref, b_ref, sel_ref, bc_ref,
                fine_ref, coarse_ref, comb_ref):
    zt = z_ref[...]
    d = zt.shape[0]

    d2e = (jnp.dot(nmu_ref[...], zt, preferred_element_type=jnp.float32)
           + jnp.dot(onesk_ref[...], zt * zt,
                     preferred_element_type=jnp.float32)
           + d2c_ref[...])
    w = mass_ref[...] / d2e

    r2 = jnp.dot(m2_ref[...], w, preferred_element_type=jnp.float32)
    fine = zt * (1.0 - r2[d:]) + r2[:d]

    cr = (jnp.dot(wt_ref[...], fine, preferred_element_type=jnp.float32)
          + b_ref[...])

    u = jnp.concatenate([fine * fine, cr * cr], axis=0)
    ss = jnp.dot(sel_ref[...], u, preferred_element_type=jnp.float32)
    inv = jax.lax.rsqrt(ss + 1e-30)
    binv = jnp.dot(bc_ref[...], inv, preferred_element_type=jnp.float32)

    comb = jnp.concatenate([fine, cr], axis=0) * binv
    fine_ref[...] = fine
    coarse_ref[...] = comb[d:]
    comb_ref[...] = comb


@jax.jit
def _run(z, mu_fine, mass_fine, W_coarse, b_coarse):
    n, d = z.shape
    k = mu_fine.shape[0]
    dc = W_coarse.shape[1]
    block = 16384
    grid = (pl.cdiv(n, block),)

    mu = mu_fine
    nmu = -2.0 * mu
    onesk = jnp.ones((k, d), jnp.float32)
    d2c = (jnp.sum(mu * mu, axis=1) + _EPS)[:, None]
    mass = mass_fine[:, None]
    m2 = jnp.concatenate([mu.T, jnp.ones((d, k), jnp.float32)], axis=0)
    sel = jnp.concatenate([
        jnp.concatenate([jnp.ones((1, d), jnp.float32),
                         jnp.zeros((1, dc), jnp.float32)], axis=1),
        jnp.concatenate([jnp.zeros((1, d), jnp.float32),
                         jnp.ones((1, dc), jnp.float32)], axis=1),
    ], axis=0)
    bc = jnp.concatenate([
        jnp.tile(jnp.array([[1.0, 0.0]], jnp.float32), (d, 1)),
        jnp.tile(jnp.array([[0.0, 1.0]], jnp.float32), (dc, 1)),
    ], axis=0)

    full = lambda shape: pl.BlockSpec(shape, lambda i: (0, 0))
    finet, coarset, combt = pl.pallas_call(
        _fused_body,
        grid=grid,
        in_specs=[
            pl.BlockSpec((d, block), lambda i: (0, i)),
            full((k, d)), full((k, d)), full((k, 1)), full((k, 1)),
            full((2 * d, k)), full((dc, d)), full((dc, 1)),
            full((2, d + dc)), full((d + dc, 2)),
        ],
        out_specs=[
            pl.BlockSpec((d, block), lambda i: (0, i)),
            pl.BlockSpec((dc, block), lambda i: (0, i)),
            pl.BlockSpec((d + dc, block), lambda i: (0, i)),
        ],
        out_shape=[
            jax.ShapeDtypeStruct((d, n), jnp.float32),
            jax.ShapeDtypeStruct((dc, n), jnp.float32),
            jax.ShapeDtypeStruct((d + dc, n), jnp.float32),
        ],
        compiler_params=pltpu.CompilerParams(
            dimension_semantics=("parallel",)),
    )(z.T, nmu, onesk, d2c, mass, m2, W_coarse.T, b_coarse[:, None],
      sel, bc)
    return finet.T, coarset.T, combt.T


def kernel(z, mu_fine, mass_fine, W_coarse, b_coarse):
    return _run(z, mu_fine, mass_fine, W_coarse, b_coarse)


# speedup vs baseline: 7.7810x; 1.0173x over previous
import functools

import jax
import jax.numpy as jnp
from jax.experimental import pallas as pl
from jax.experimental.pallas import tpu as pltpu

_EPS = 1.0


def _fused_body(z_ref, nmu_ref, onesk_ref, d2c_ref, mass_ref, m2_ref,
                wt_ref, b_ref, sel_ref, bc_ref,
                fine_ref, coarse_ref, comb_ref):
    zt = z_ref[...]
    d = zt.shape[0]

    d2e = (jnp.dot(nmu_ref[...], zt, preferred_element_type=jnp.float32)
           + jnp.dot(onesk_ref[...], zt * zt,
                     preferred_element_type=jnp.float32)
           + d2c_ref[...])
    w = mass_ref[...] / d2e

    r2 = jnp.dot(m2_ref[...], w, preferred_element_type=jnp.float32)
    fine = zt * (1.0 - r2[d:]) + r2[:d]

    cr = (jnp.dot(wt_ref[...], fine, preferred_element_type=jnp.float32)
          + b_ref[...])

    u = jnp.concatenate([fine * fine, cr * cr], axis=0)
    ss = jnp.dot(sel_ref[...], u, preferred_element_type=jnp.float32)
    inv = jax.lax.rsqrt(ss + 1e-30)
    binv = jnp.dot(bc_ref[...], inv, preferred_element_type=jnp.float32)

    comb = jnp.concatenate([fine, cr], axis=0) * binv
    fine_ref[...] = fine
    coarse_ref[...] = comb[d:]
    comb_ref[...] = comb


@jax.jit
def _run(z, mu_fine, mass_fine, W_coarse, b_coarse):
    n, d = z.shape
    k = mu_fine.shape[0]
    dc = W_coarse.shape[1]
    block = 24576
    grid = (pl.cdiv(n, block),)

    mu = mu_fine
    nmu = -2.0 * mu
    onesk = jnp.ones((k, d), jnp.float32)
    d2c = (jnp.sum(mu * mu, axis=1) + _EPS)[:, None]
    mass = mass_fine[:, None]
    m2 = jnp.concatenate([mu.T, jnp.ones((d, k), jnp.float32)], axis=0)
    sel = jnp.concatenate([
        jnp.concatenate([jnp.ones((1, d), jnp.float32),
                         jnp.zeros((1, dc), jnp.float32)], axis=1),
        jnp.concatenate([jnp.zeros((1, d), jnp.float32),
                         jnp.ones((1, dc), jnp.float32)], axis=1),
    ], axis=0)
    bc = jnp.concatenate([
        jnp.tile(jnp.array([[1.0, 0.0]], jnp.float32), (d, 1)),
        jnp.tile(jnp.array([[0.0, 1.0]], jnp.float32), (dc, 1)),
    ], axis=0)

    full = lambda shape: pl.BlockSpec(shape, lambda i: (0, 0))
    finet, coarset, combt = pl.pallas_call(
        _fused_body,
        grid=grid,
        in_specs=[
            pl.BlockSpec((d, block), lambda i: (0, i)),
            full((k, d)), full((k, d)), full((k, 1)), full((k, 1)),
            full((2 * d, k)), full((dc, d)), full((dc, 1)),
            full((2, d + dc)), full((d + dc, 2)),
        ],
        out_specs=[
            pl.BlockSpec((d, block), lambda i: (0, i)),
            pl.BlockSpec((dc, block), lambda i: (0, i)),
            pl.BlockSpec((d + dc, block), lambda i: (0, i)),
        ],
        out_shape=[
            jax.ShapeDtypeStruct((d, n), jnp.float32),
            jax.ShapeDtypeStruct((dc, n), jnp.float32),
            jax.ShapeDtypeStruct((d + dc, n), jnp.float32),
        ],
        compiler_params=pltpu.CompilerParams(
            dimension_semantics=("parallel",)),
    )(z.T, nmu, onesk, d2c, mass, m2, W_coarse.T, b_coarse[:, None],
      sel, bc)
    return finet.T, coarset.T, combt.T


def kernel(z, mu_fine, mass_fine, W_coarse, b_coarse):
    return _run(z, mu_fine, mass_fine, W_coarse, b_coarse)


# speedup vs baseline: 7.8278x; 1.0060x over previous
import functools

import jax
import jax.numpy as jnp
from jax.experimental import pallas as pl
from jax.experimental.pallas import tpu as pltpu

_EPS = 1.0


def _fused_body(z_ref, nmu_ref, onesk_ref, d2c_ref, mass_ref, mut_ref,
                ones1k_ref, wt_ref, b_ref, ones1d_ref, ones1dc_ref,
                fine_ref, coarse_ref, comb_ref):
    zt = z_ref[...]
    d = zt.shape[0]

    d2e = (jnp.dot(nmu_ref[...], zt, preferred_element_type=jnp.float32)
           + jnp.dot(onesk_ref[...], zt * zt,
                     preferred_element_type=jnp.float32)
           + d2c_ref[...])
    w = mass_ref[...] / d2e

    wmu = jnp.dot(mut_ref[...], w, preferred_element_type=jnp.float32)
    sw = jnp.dot(ones1k_ref[...], w, preferred_element_type=jnp.float32)
    fine = zt * (1.0 - sw) + wmu

    cr = (jnp.dot(wt_ref[...], fine, preferred_element_type=jnp.float32)
          + b_ref[...])

    ssf = jnp.dot(ones1d_ref[...], fine * fine,
                  preferred_element_type=jnp.float32)
    ssc = jnp.dot(ones1dc_ref[...], cr * cr,
                  preferred_element_type=jnp.float32)
    inv_f = jax.lax.rsqrt(ssf + 1e-30)
    inv_c = jax.lax.rsqrt(ssc + 1e-30)

    comb = jnp.concatenate([fine * inv_f, cr * inv_c], axis=0)
    fine_ref[...] = fine
    coarse_ref[...] = comb[d:]
    comb_ref[...] = comb


@jax.jit
def _run(z, mu_fine, mass_fine, W_coarse, b_coarse):
    n, d = z.shape
    k = mu_fine.shape[0]
    dc = W_coarse.shape[1]
    block = 24576
    grid = (pl.cdiv(n, block),)

    mu = mu_fine
    nmu = -2.0 * mu
    onesk = jnp.ones((k, d), jnp.float32)
    d2c = (jnp.sum(mu * mu, axis=1) + _EPS)[:, None]
    mass = mass_fine[:, None]
    mut = mu.T
    ones1k = jnp.ones((1, k), jnp.float32)
    ones1d = jnp.ones((1, d), jnp.float32)
    ones1dc = jnp.ones((1, dc), jnp.float32)

    full = lambda shape: pl.BlockSpec(shape, lambda i: (0, 0))
    finet, coarset, combt = pl.pallas_call(
        _fused_body,
        grid=grid,
        in_specs=[
            pl.BlockSpec((d, block), lambda i: (0, i)),
            full((k, d)), full((k, d)), full((k, 1)), full((k, 1)),
            full((d, k)), full((1, k)), full((dc, d)), full((dc, 1)),
            full((1, d)), full((1, dc)),
        ],
        out_specs=[
            pl.BlockSpec((d, block), lambda i: (0, i)),
            pl.BlockSpec((dc, block), lambda i: (0, i)),
            pl.BlockSpec((d + dc, block), lambda i: (0, i)),
        ],
        out_shape=[
            jax.ShapeDtypeStruct((d, n), jnp.float32),
            jax.ShapeDtypeStruct((dc, n), jnp.float32),
            jax.ShapeDtypeStruct((d + dc, n), jnp.float32),
        ],
        compiler_params=pltpu.CompilerParams(
            dimension_semantics=("parallel",)),
    )(z.T, nmu, onesk, d2c, mass, mut, ones1k, W_coarse.T, b_coarse[:, None],
      ones1d, ones1dc)
    return finet.T, coarset.T, combt.T


def kernel(z, mu_fine, mass_fine, W_coarse, b_coarse):
    return _run(z, mu_fine, mass_fine, W_coarse, b_coarse)


# speedup vs baseline: 7.8492x; 1.0027x over previous
import functools

import jax
import jax.numpy as jnp
from jax.experimental import pallas as pl
from jax.experimental.pallas import tpu as pltpu

_EPS = 1.0


def _fused_body(z_ref, nmu_ref, onesk_ref, d2c_ref, mass_ref, mut_ref,
                ones1k_ref, wt_ref, b_ref, ones1d_ref, ones1dc_ref,
                fine_ref, coarse_ref, comb_ref):
    zt = z_ref[...]
    d = zt.shape[0]

    d2e = (jnp.dot(nmu_ref[...], zt, preferred_element_type=jnp.float32)
           + jnp.dot(onesk_ref[...], zt * zt,
                     preferred_element_type=jnp.float32)
           + d2c_ref[...])
    w = mass_ref[...] / d2e

    wmu = jnp.dot(mut_ref[...], w, preferred_element_type=jnp.float32)
    sw = jnp.dot(ones1k_ref[...], w, preferred_element_type=jnp.float32)
    fine = zt * (1.0 - sw) + wmu

    cr = (jnp.dot(wt_ref[...], fine, preferred_element_type=jnp.float32)
          + b_ref[...])

    ssf = jnp.dot(ones1d_ref[...], fine * fine,
                  preferred_element_type=jnp.float32)
    ssc = jnp.dot(ones1dc_ref[...], cr * cr,
                  preferred_element_type=jnp.float32)
    inv_f = jax.lax.rsqrt(ssf + 1e-30)
    inv_c = jax.lax.rsqrt(ssc + 1e-30)

    comb = jnp.concatenate([fine * inv_f, cr * inv_c], axis=0)
    fine_ref[...] = fine
    coarse_ref[...] = comb[d:]
    comb_ref[...] = comb


@jax.jit
def _run(z, mu_fine, mass_fine, W_coarse, b_coarse):
    n, d = z.shape
    k = mu_fine.shape[0]
    dc = W_coarse.shape[1]
    block = 28672
    grid = (pl.cdiv(n, block),)

    mu = mu_fine
    nmu = -2.0 * mu
    onesk = jnp.ones((k, d), jnp.float32)
    d2c = (jnp.sum(mu * mu, axis=1) + _EPS)[:, None]
    mass = mass_fine[:, None]
    mut = mu.T
    ones1k = jnp.ones((1, k), jnp.float32)
    ones1d = jnp.ones((1, d), jnp.float32)
    ones1dc = jnp.ones((1, dc), jnp.float32)

    full = lambda shape: pl.BlockSpec(shape, lambda i: (0, 0))
    finet, coarset, combt = pl.pallas_call(
        _fused_body,
        grid=grid,
        in_specs=[
            pl.BlockSpec((d, block), lambda i: (0, i)),
            full((k, d)), full((k, d)), full((k, 1)), full((k, 1)),
            full((d, k)), full((1, k)), full((dc, d)), full((dc, 1)),
            full((1, d)), full((1, dc)),
        ],
        out_specs=[
            pl.BlockSpec((d, block), lambda i: (0, i)),
            pl.BlockSpec((dc, block), lambda i: (0, i)),
            pl.BlockSpec((d + dc, block), lambda i: (0, i)),
        ],
        out_shape=[
            jax.ShapeDtypeStruct((d, n), jnp.float32),
            jax.ShapeDtypeStruct((dc, n), jnp.float32),
            jax.ShapeDtypeStruct((d + dc, n), jnp.float32),
        ],
        compiler_params=pltpu.CompilerParams(
            dimension_semantics=("parallel",)),
    )(z.T, nmu, onesk, d2c, mass, mut, ones1k, W_coarse.T, b_coarse[:, None],
      ones1d, ones1dc)
    return finet.T, coarset.T, combt.T


def kernel(z, mu_fine, mass_fine, W_coarse, b_coarse):
    return _run(z, mu_fine, mass_fine, W_coarse, b_coarse)
